# baseline (speedup 1.0000x reference)
import functools
import jax
import jax.numpy as jnp
from jax import lax
from jax.experimental import pallas as pl
from jax.experimental.pallas import tpu as pltpu
from jax.experimental.pallas import tpu_sc as plsc

B_DIM = 4096
T_DIM = 200
D = 64
V2 = 500000
SCALE = 8.0

NUM_CORES = 2
NUM_SUBCORES = 16
NW = NUM_CORES * NUM_SUBCORES
BBLK = B_DIM // NW
NBUF = 2

_mesh = plsc.VectorSubcoreMesh(core_axis_name="c", subcore_axis_name="s")


@functools.partial(
    pl.kernel,
    mesh=_mesh,
    out_type=jax.ShapeDtypeStruct((T_DIM, D, B_DIM), jnp.float32),
    scratch_types=[
        pltpu.VMEM((T_DIM, BBLK), jnp.int32),
        *[pltpu.VMEM((BBLK,), jnp.int32) for _ in range(NBUF)],
        *[pltpu.VMEM((BBLK, 128), jnp.float32) for _ in range(NBUF)],
        *[pltpu.VMEM((D, BBLK), jnp.float32) for _ in range(NBUF)],
        *[pltpu.SemaphoreType.DMA for _ in range(2 * NBUF)],
    ],
    compiler_params=pltpu.CompilerParams(
        use_tc_tiling_on_sc=True, needs_layout_passes=False),
)
def _emb_lookup(xt_hbm, t2_hbm, out_hbm, idx_v, *scr):
    jbuf = scr[:NBUF]
    buf = scr[NBUF:2 * NBUF]
    obuf = scr[2 * NBUF:3 * NBUF]
    gsem = scr[3 * NBUF:4 * NBUF]
    wsem = scr[4 * NBUF:]
    wid = lax.axis_index("s") * NUM_CORES + lax.axis_index("c")
    col0 = wid * BBLK

    pltpu.sync_copy(xt_hbm.at[:, pl.ds(col0, BBLK)], idx_v)

    def compute_jbuf(t, b):
        for s in range(BBLK // 16):
            sl = pl.ds(16 * s, 16)
            jbuf[b][sl] = lax.shift_right_logical(idx_v[t, sl], 1)

    def start_gather(b):
        pltpu.make_async_copy(t2_hbm.at[jbuf[b]], buf[b], gsem[b]).start()

    def wait_gather(b):
        pltpu.make_async_copy(t2_hbm.at[jbuf[b]], buf[b], gsem[b]).wait()

    def start_write(t, b):
        pltpu.make_async_copy(
            obuf[b], out_hbm.at[t, :, pl.ds(col0, BBLK)], wsem[b]).start()

    def wait_write(t, b):
        pltpu.make_async_copy(
            obuf[b], out_hbm.at[t, :, pl.ds(col0, BBLK)], wsem[b]).wait()

    def compute_obuf(t, b):
        def sbody(s, carry):
            sl = pl.ds(16 * s, 16)
            idxv = idx_v[t, sl]
            rowv = lax.iota(jnp.int32, 16) + (16 * s)
            colv0 = (idxv & 1) * D

            @plsc.parallel_loop(0, D, unroll=8)
            def dbody(d):
                val = plsc.load_gather(buf[b], [rowv, colv0 + d])
                obuf[b][d, sl] = val * SCALE

            return carry

        lax.fori_loop(0, BBLK // 16, sbody, 0)

    for b in range(NBUF):
        compute_jbuf(b, b)
        start_gather(b)

    def body(g, carry):
        t0 = g * NBUF
        for b in range(NBUF):
            wait_gather(b)
            compute_obuf(t0 + b, b)
            start_write(t0 + b, b)
        for b in range(NBUF):
            wait_write(t0 + b, b)
            compute_jbuf(t0 + NBUF + b, b)
            start_gather(b)
        return carry

    lax.fori_loop(0, T_DIM // NBUF - 1, body, 0)

    tlast = T_DIM - NBUF
    for b in range(NBUF):
        wait_gather(b)
        compute_obuf(tlast + b, b)
        start_write(tlast + b, b)
    for b in range(NBUF):
        wait_write(tlast + b, b)


def kernel(x, table):
    xt = x.T.astype(jnp.int32)
    t2 = jnp.concatenate([table[0::2], table[1::2]], axis=1)
    out_t = _emb_lookup(xt, t2)
    return out_t.transpose(2, 0, 1)


# speedup vs baseline: 6.9449x; 6.9449x over previous
import functools
import jax
import jax.numpy as jnp
from jax import lax
from jax.experimental import pallas as pl
from jax.experimental.pallas import tpu as pltpu
from jax.experimental.pallas import tpu_sc as plsc

B_DIM = 4096
T_DIM = 200
D = 64
V2 = 500000
SCALE = 8.0

NUM_CORES = 2
NUM_SUBCORES = 16
NW = NUM_CORES * NUM_SUBCORES
BBLK = B_DIM // NW
NBUF = 2

_mesh = plsc.VectorSubcoreMesh(core_axis_name="c", subcore_axis_name="s")


@functools.partial(
    pl.kernel,
    mesh=_mesh,
    out_type=jax.ShapeDtypeStruct((T_DIM, D, B_DIM), jnp.float32),
    scratch_types=[
        pltpu.VMEM((T_DIM, BBLK), jnp.int32),
        *[pltpu.VMEM((BBLK,), jnp.int32) for _ in range(NBUF)],
        *[pltpu.VMEM((BBLK, 128), jnp.float32) for _ in range(NBUF)],
        *[pltpu.VMEM((D, BBLK), jnp.float32) for _ in range(NBUF)],
        *[pltpu.SemaphoreType.DMA for _ in range(2 * NBUF)],
    ],
    compiler_params=pltpu.CompilerParams(
        use_tc_tiling_on_sc=True, needs_layout_passes=False),
)
def _emb_lookup(xt_hbm, t2_hbm, out_hbm, idx_v, *scr):
    jbuf = scr[:NBUF]
    buf = scr[NBUF:2 * NBUF]
    obuf = scr[2 * NBUF:3 * NBUF]
    gsem = scr[3 * NBUF:4 * NBUF]
    wsem = scr[4 * NBUF:]
    wid = lax.axis_index("s") * NUM_CORES + lax.axis_index("c")
    col0 = wid * BBLK

    pltpu.sync_copy(xt_hbm.at[:, pl.ds(col0, BBLK)], idx_v)

    def compute_jbuf(t, b):
        for s in range(BBLK // 16):
            sl = pl.ds(16 * s, 16)
            jbuf[b][sl] = lax.shift_right_logical(idx_v[t, sl], 1)

    def start_gather(b):
        pltpu.make_async_copy(t2_hbm.at[jbuf[b]], buf[b], gsem[b]).start()

    def wait_gather(b):
        pltpu.make_async_copy(t2_hbm.at[jbuf[b]], buf[b], gsem[b]).wait()

    def start_write(t, b):
        pltpu.make_async_copy(
            obuf[b], out_hbm.at[t, :, pl.ds(col0, BBLK)], wsem[b]).start()

    def wait_write(t, b):
        pltpu.make_async_copy(
            obuf[b], out_hbm.at[t, :, pl.ds(col0, BBLK)], wsem[b]).wait()

    def compute_obuf(t, b):
        iota = lax.iota(jnp.int32, 16)

        @plsc.parallel_loop(0, BBLK // 16, unroll=1)
        def gbody(g):
            bbase = 16 * g
            sl = pl.ds(bbase, 16)
            idxv = idx_v[t, sl]
            rowv = bbase + iota
            colbase = (idxv & 1) * D
            for q in range(D // 16):
                for k in range(16):
                    rot = (iota + k) & 15
                    colv = colbase + (16 * q) + rot
                    val = plsc.load_gather(buf[b], [rowv, colv])
                    plsc.store_scatter(
                        obuf[b], [(16 * q) + rot, rowv], val * SCALE)

    for b in range(NBUF):
        compute_jbuf(b, b)
        start_gather(b)

    def body(g, carry):
        t0 = g * NBUF
        for b in range(NBUF):
            wait_gather(b)
            compute_obuf(t0 + b, b)
            start_write(t0 + b, b)
        for b in range(NBUF):
            wait_write(t0 + b, b)
            compute_jbuf(t0 + NBUF + b, b)
            start_gather(b)
        return carry

    lax.fori_loop(0, T_DIM // NBUF - 1, body, 0)

    tlast = T_DIM - NBUF
    for b in range(NBUF):
        wait_gather(b)
        compute_obuf(tlast + b, b)
        start_write(tlast + b, b)
    for b in range(NBUF):
        wait_write(tlast + b, b)


def kernel(x, table):
    xt = x.T.astype(jnp.int32)
    t2 = table.reshape(V2, 128)
    out_t = _emb_lookup(xt, t2)
    return out_t.transpose(2, 0, 1)


# speedup vs baseline: 7.7869x; 1.1212x over previous
import functools
import jax
import jax.numpy as jnp
from jax import lax
from jax.experimental import pallas as pl
from jax.experimental.pallas import tpu as pltpu
from jax.experimental.pallas import tpu_sc as plsc

B_ROWS = 4096
B_COLS = 200
D = 64
SCALE = 8.0

NUM_CORES = 2
NUM_SUBCORES = 16
NW = NUM_CORES * NUM_SUBCORES
ROWS_PER_W = B_ROWS // NW
NBUF = 4
ROW_UNROLL = 4

_mesh = plsc.VectorSubcoreMesh(core_axis_name="c", subcore_axis_name="s")


@functools.partial(
    pl.kernel,
    mesh=_mesh,
    out_type=jax.ShapeDtypeStruct((B_ROWS, B_COLS, D), jnp.float32),
    scratch_types=[
        pltpu.VMEM((ROWS_PER_W, B_COLS), jnp.int32),
        *[pltpu.VMEM((B_COLS, D), jnp.float32) for _ in range(NBUF)],
        *[pltpu.SemaphoreType.DMA for _ in range(2 * NBUF)],
    ],
    compiler_params=pltpu.CompilerParams(use_tc_tiling_on_sc=False),
)
def _emb_lookup(idx_hbm, table_hbm, out_hbm, idx_v, *bufs_and_sems):
    rows = bufs_and_sems[:NBUF]
    gsem = bufs_and_sems[NBUF:2 * NBUF]
    wsem = bufs_and_sems[2 * NBUF:]
    wid = lax.axis_index("s") * NUM_CORES + lax.axis_index("c")
    row0 = wid * ROWS_PER_W

    pltpu.sync_copy(idx_hbm.at[pl.ds(row0, ROWS_PER_W), :], idx_v)

    def start_gather(r, b):
        pltpu.make_async_copy(table_hbm.at[idx_v.at[r]], rows[b], gsem[b]).start()

    def wait_gather(r, b):
        pltpu.make_async_copy(table_hbm.at[idx_v.at[r]], rows[b], gsem[b]).wait()

    def start_write(r, b):
        pltpu.make_async_copy(rows[b], out_hbm.at[row0 + r], wsem[b]).start()

    def wait_write(r, b):
        pltpu.make_async_copy(rows[b], out_hbm.at[row0 + r], wsem[b]).wait()

    def scale_buf(b):
        def scale_rows(r, carry):
            q0 = r * ROW_UNROLL
            for dq in range(ROW_UNROLL):
                for jj in range(D // 16):
                    sl = pl.ds(16 * jj, 16)
                    rows[b][q0 + dq, sl] = rows[b][q0 + dq, sl] * SCALE
            return carry

        lax.fori_loop(0, B_COLS // ROW_UNROLL, scale_rows, 0)

    for b in range(NBUF):
        start_gather(b, b)

    def body(g, carry):
        j0 = g * NBUF
        for b in range(NBUF):
            wait_gather(j0 + b, b)
            scale_buf(b)
            start_write(j0 + b, b)
        for b in range(NBUF):
            wait_write(j0 + b, b)
            start_gather(j0 + NBUF + b, b)
        return carry

    lax.fori_loop(0, ROWS_PER_W // NBUF - 1, body, 0)

    jlast = ROWS_PER_W - NBUF
    for b in range(NBUF):
        wait_gather(jlast + b, b)
        scale_buf(b)
        start_write(jlast + b, b)
    for b in range(NBUF):
        wait_write(jlast + b, b)


def kernel(x, table):
    return _emb_lookup(x.astype(jnp.int32), table)
